# Initial kernel scaffold; baseline (speedup 1.0000x reference)
import jax
import jax.numpy as jnp
from jax.experimental import pallas as pl


def kernel(input_ids, params):
    raise NotImplementedError("write your pallas kernel here")


# speedup vs baseline: 1.3328x; 1.3328x over previous
import math

import jax
import jax.numpy as jnp
from jax.experimental import pallas as pl
from jax.experimental.pallas import tpu as pltpu

D_MODEL = 768
N_HEADS = 12
DH = 64
IDX_HEADS = 4
IDX_DIM = 64
TOPK = 64
SEQ = 2048
VOCAB = 32000
BQ = 256


def _embed(ids, tok, pos):
    def body(ids_ref, tok_ref, pos_ref, out_ref):
        out_ref[...] = tok_ref[...] + pos_ref[...]

    grid_spec = pltpu.PrefetchScalarGridSpec(
        num_scalar_prefetch=1,
        grid=(SEQ,),
        in_specs=[
            pl.BlockSpec((1, 1, D_MODEL), lambda i, ids: (ids[i], 0, 0)),
            pl.BlockSpec((1, 1, D_MODEL), lambda i, ids: (i, 0, 0)),
        ],
        out_specs=pl.BlockSpec((1, 1, D_MODEL), lambda i, ids: (i, 0, 0)),
    )
    out = pl.pallas_call(
        body, grid_spec=grid_spec,
        out_shape=jax.ShapeDtypeStruct((SEQ, 1, D_MODEL), jnp.float32))(
            ids, tok.reshape(VOCAB, 1, D_MODEL), pos.reshape(SEQ, 1, D_MODEL))
    return out.reshape(SEQ, D_MODEL)


def _matmul(x, w, *, bm, bn, ln=None, bias=None, act=None, res=None):
    M, K = x.shape
    N = w.shape[1]
    inputs = [x, w]
    in_specs = [
        pl.BlockSpec((bm, K), lambda i, j: (i, 0)),
        pl.BlockSpec((K, bn), lambda i, j: (0, j)),
    ]
    n_fixed = 2
    if ln is not None:
        s, b = ln
        inputs += [s.reshape(1, K), b.reshape(1, K)]
        in_specs += [pl.BlockSpec((1, K), lambda i, j: (0, 0))] * 2
        n_fixed += 2
    if bias is not None:
        inputs.append(bias.reshape(1, N))
        in_specs.append(pl.BlockSpec((1, bn), lambda i, j: (0, j)))
    if res is not None:
        inputs.append(res)
        in_specs.append(pl.BlockSpec((bm, bn), lambda i, j: (i, j)))

    def body(*refs):
        out_ref = refs[-1]
        xb = refs[0][...]
        wb = refs[1][...]
        idx = 2
        if ln is not None:
            sb = refs[2][...]
            bb = refs[3][...]
            idx = 4
            m = jnp.mean(xb, axis=1, keepdims=True)
            v = jnp.mean((xb - m) ** 2, axis=1, keepdims=True)
            xb = (xb - m) / jnp.sqrt(v + 1e-5) * sb + bb
        y = jnp.dot(xb.astype(jnp.bfloat16), wb.astype(jnp.bfloat16),
                    preferred_element_type=jnp.float32)
        if bias is not None:
            y = y + refs[idx][...]
            idx += 1
        if act == 'gelu':
            c = math.sqrt(2.0 / math.pi)
            y = 0.5 * y * (1.0 + jnp.tanh(c * (y + 0.044715 * (y ** 3))))
        if res is not None:
            y = y + refs[idx][...]
            idx += 1
        out_ref[...] = y

    return pl.pallas_call(
        body, grid=(M // bm, N // bn), in_specs=in_specs,
        out_specs=pl.BlockSpec((bm, bn), lambda i, j: (i, j)),
        out_shape=jax.ShapeDtypeStruct((M, N), jnp.float32))(*inputs)


def _topk_bias(qi, kiT, wi_pad):
    def body(qi_ref, kiT_ref, wi_ref, bias_ref, work_ref):
        i = pl.program_id(0)
        qi_b = qi_ref[...]
        kiT_b = kiT_ref[...]
        wi_b = wi_ref[...]
        acc = jnp.zeros((BQ, SEQ), jnp.float32)
        for h in range(IDX_HEADS):
            sh = jnp.dot(qi_b[:, h * IDX_DIM:(h + 1) * IDX_DIM].astype(jnp.bfloat16),
                         kiT_b.astype(jnp.bfloat16), preferred_element_type=jnp.float32)
            sh = jnp.maximum(sh, 0.0).astype(jnp.bfloat16).astype(jnp.float32)
            wh = wi_b[:, h:h + 1].astype(jnp.bfloat16).astype(jnp.float32)
            acc = acc + sh * wh
        col = jax.lax.broadcasted_iota(jnp.int32, (BQ, SEQ), 1)
        row = jax.lax.broadcasted_iota(jnp.int32, (BQ, SEQ), 0) + i * BQ
        causal = col <= row
        ninf = jnp.float32(-jnp.inf)
        work_ref[...] = jnp.where(causal, acc, ninf)

        def it(_, carry):
            wk = work_ref[...]
            m = jnp.max(wk, axis=1, keepdims=True)
            eq = wk == m
            first = jnp.min(jnp.where(eq, col, SEQ), axis=1, keepdims=True)
            work_ref[...] = jnp.where(col == first, ninf, wk)
            return carry

        jax.lax.fori_loop(0, TOPK, it, 0)
        sel = causal & (work_ref[...] == ninf)
        bias_ref[...] = jnp.where(sel, 0.0, ninf)

    return pl.pallas_call(
        body, grid=(SEQ // BQ,),
        in_specs=[
            pl.BlockSpec((BQ, IDX_HEADS * IDX_DIM), lambda i: (i, 0)),
            pl.BlockSpec((IDX_DIM, SEQ), lambda i: (0, 0)),
            pl.BlockSpec((BQ, 128), lambda i: (i, 0)),
        ],
        out_specs=pl.BlockSpec((BQ, SEQ), lambda i: (i, 0)),
        scratch_shapes=[pltpu.VMEM((BQ, SEQ), jnp.float32)],
        out_shape=jax.ShapeDtypeStruct((SEQ, SEQ), jnp.float32))(qi, kiT, wi_pad)


def _attention(Qh, Kt, V, bias):
    scale = 1.0 / math.sqrt(DH)

    def body(q_ref, k_ref, v_ref, b_ref, o_ref):
        q = q_ref[0]
        k = k_ref[0]
        v = v_ref[0]
        b = b_ref[...]
        s = jnp.dot(q.astype(jnp.bfloat16), k.astype(jnp.bfloat16),
                    preferred_element_type=jnp.float32) * scale + b
        m = jnp.max(s, axis=1, keepdims=True)
        p = jnp.exp(s - m)
        l = jnp.sum(p, axis=1, keepdims=True)
        o = jnp.dot(p.astype(jnp.bfloat16), v.astype(jnp.bfloat16),
                    preferred_element_type=jnp.float32)
        o_ref[0] = o / l

    return pl.pallas_call(
        body, grid=(SEQ // BQ, N_HEADS),
        in_specs=[
            pl.BlockSpec((1, BQ, DH), lambda i, h: (h, i, 0)),
            pl.BlockSpec((1, DH, SEQ), lambda i, h: (h, 0, 0)),
            pl.BlockSpec((1, SEQ, DH), lambda i, h: (h, 0, 0)),
            pl.BlockSpec((BQ, SEQ), lambda i, h: (i, 0)),
        ],
        out_specs=pl.BlockSpec((1, BQ, DH), lambda i, h: (h, i, 0)),
        out_shape=jax.ShapeDtypeStruct((N_HEADS, SEQ, DH), jnp.float32))(Qh, Kt, V, bias)


def _rope2d(x, cos_rep, sin_rep):
    r = x.reshape(SEQ, D_MODEL // 2, 2)
    rot = jnp.stack([-r[..., 1], r[..., 0]], -1).reshape(SEQ, D_MODEL)
    return x * cos_rep + rot * sin_rep


def _lnorm(x, s, b):
    m = x.mean(-1, keepdims=True)
    v = ((x - m) ** 2).mean(-1, keepdims=True)
    return (x - m) / jnp.sqrt(v + 1e-5) * s + b


def _layer(x, lp, cos_rep, sin_rep):
    Wcat = jnp.concatenate(
        [lp['Wqkv'], lp['Wiq'], lp['Wik'], lp['Wiw'],
         jnp.zeros((D_MODEL, 60), jnp.float32)], axis=1)
    h = _matmul(_lnorm(x, lp['ln1_s'], lp['ln1_b']), Wcat, bm=1024, bn=384)
    q = h[:, :768]
    k = h[:, 768:1536]
    v = h[:, 1536:2304]
    qi = h[:, 2304:2560]
    ki = h[:, 2560:2624]
    wi = h[:, 2624:2628]
    qr = _rope2d(q, cos_rep, sin_rep)
    kr = _rope2d(k, cos_rep, sin_rep)
    Qh = qr.reshape(SEQ, N_HEADS, DH).transpose(1, 0, 2)
    Kt = kr.reshape(SEQ, N_HEADS, DH).transpose(1, 2, 0)
    V = v.reshape(SEQ, N_HEADS, DH).transpose(1, 0, 2)
    bias = _topk_bias(qi, ki.T, jnp.pad(wi, ((0, 0), (0, 124))))
    attn = _attention(Qh, Kt, V, bias)
    attn = attn.transpose(1, 0, 2).reshape(SEQ, D_MODEL)
    x = _matmul(attn, lp['Wo'], bm=1024, bn=384, res=x)
    hh = _matmul(_lnorm(x, lp['ln2_s'], lp['ln2_b']), lp['W1'], bm=1024, bn=384,
                 bias=lp['b1'])
    hh = jax.nn.gelu(hh)
    x = _matmul(hh, lp['W2'], bm=256, bn=384, bias=lp['b2'], res=x)
    return x


def kernel(input_ids, params):
    ids = input_ids.reshape(-1).astype(jnp.int32)
    half = DH // 2
    inv = 1.0 / (10000.0 ** (jnp.arange(half, dtype=jnp.float32) * 2.0 / DH))
    ang = jnp.arange(SEQ, dtype=jnp.float32)[:, None] * inv[None, :]
    cos_rep = jnp.tile(jnp.repeat(jnp.cos(ang), 2, axis=1), (1, N_HEADS))
    sin_rep = jnp.tile(jnp.repeat(jnp.sin(ang), 2, axis=1), (1, N_HEADS))
    x = _embed(ids, params['tok'], params['pos'])
    for lp in params['layers']:
        x = _layer(x, lp, cos_rep, sin_rep)
    logits = _matmul(_lnorm(x, params['lnf_s'], params['lnf_b']),
                     params['lm_head'], bm=2048, bn=256)
    return logits.reshape(1, SEQ, VOCAB)
